# Initial kernel scaffold; baseline (speedup 1.0000x reference)
import jax
import jax.numpy as jnp
from jax.experimental import pallas as pl


def kernel(x, pos, batch, tq, c1_W1, c1_b1, c1_W2, c1_b2, c2_W1, c2_b1, c2_W2, c2_b2, l1_W1, l1_b1, l1_W2, l1_b2, m_W1, m_b1, m_W2, m_b2):
    raise NotImplementedError("write your pallas kernel here")


# speedup vs baseline: 14.9300x; 14.9300x over previous
import functools
import jax
import jax.numpy as jnp
from jax import lax
from jax.experimental import pallas as pl
from jax.experimental.pallas import tpu as pltpu
from jax.experimental.pallas import tpu_sc as plsc

NP_ = 1024
K_ = 16


def _leaky(v):
    return jnp.where(v >= 0, v, 0.01 * v)


def _prep_body(xin_ref, w1a_ref, w1b_ref, b1_ref, idx_ref, a_ref, bm_ref):
    b = pl.program_id(0)
    x = xin_ref[0]
    F = w1a_ref.shape[1]
    pad = jnp.zeros((NP_, 128 - F), jnp.float32) if F < 128 else None
    a = jnp.dot(x, w1a_ref[...], preferred_element_type=jnp.float32) + b1_ref[0]
    bm = jnp.dot(x, w1b_ref[...], preferred_element_type=jnp.float32)
    if pad is not None:
        a = jnp.concatenate([a, pad], axis=1)
        bm = jnp.concatenate([bm, pad], axis=1)
    a_ref[0] = a
    bm_ref[0] = bm
    sq = jnp.sum(x * x, axis=1)
    xxt = lax.dot_general(x, x, (((1,), (1,)), ((), ())),
                          preferred_element_type=jnp.float32)
    d2 = sq[:, None] + sq[None, :] - 2.0 * xxt
    iota = lax.broadcasted_iota(jnp.int32, (NP_, NP_), 1)
    base = b * NP_
    cols = []
    for _ in range(K_):
        m = jnp.min(d2, axis=1)
        sel = jnp.where(d2 <= m[:, None], iota, NP_)
        idxsel = jnp.min(sel, axis=1)
        cols.append(idxsel + base)
        d2 = jnp.where(iota == idxsel[:, None], jnp.float32(jnp.inf), d2)
    idx_ref[0] = jnp.stack(cols, axis=0)


def _prep(xin, w1a, w1b, b1):
    B = xin.shape[0]
    d = xin.shape[2]
    F = w1a.shape[1]
    return pl.pallas_call(
        _prep_body,
        grid=(B,),
        in_specs=[
            pl.BlockSpec((1, NP_, d), lambda b: (b, 0, 0)),
            pl.BlockSpec((d, F), lambda b: (0, 0)),
            pl.BlockSpec((d, F), lambda b: (0, 0)),
            pl.BlockSpec((1, F), lambda b: (0, 0)),
        ],
        out_specs=[
            pl.BlockSpec((1, K_, NP_), lambda b: (b, 0, 0)),
            pl.BlockSpec((1, NP_, 128), lambda b: (b, 0, 0)),
            pl.BlockSpec((1, NP_, 128), lambda b: (b, 0, 0)),
        ],
        out_shape=[
            jax.ShapeDtypeStruct((B, K_, NP_), jnp.int32),
            jax.ShapeDtypeStruct((B, NP_, 128), jnp.float32),
            jax.ShapeDtypeStruct((B, NP_, 128), jnp.float32),
        ],
    )(xin, w1a, w1b, b1)


_CH = 128
_NBUF = 4


def _gather_rows(table, idx):
    R, F = table.shape
    E = idx.shape[0]
    C = E // _CH
    NW = 32
    cpw = C // NW
    idx2d = idx.reshape(C, _CH)
    mesh = plsc.VectorSubcoreMesh(core_axis_name="c", subcore_axis_name="s")

    @functools.partial(
        pl.kernel,
        out_type=jax.ShapeDtypeStruct((C, _CH, F), jnp.float32),
        mesh=mesh,
        scratch_types=[
            pltpu.VMEM((cpw, _CH), jnp.int32),
            *[pltpu.VMEM((_CH, F), jnp.float32) for _ in range(_NBUF)],
            pltpu.SemaphoreType.DMA,
            pltpu.SemaphoreType.DMA,
        ],
    )
    def k(table_hbm, idx_hbm, out_hbm, idx_v, b0, b1, b2, b3, gsem, osem):
        bufs = (b0, b1, b2, b3)
        wid = lax.axis_index("s") * 2 + lax.axis_index("c")
        base = wid * cpw
        pltpu.sync_copy(idx_hbm.at[pl.ds(base, cpw)], idx_v)

        def round_(i, _):
            j0 = i * _NBUF
            for b in range(_NBUF):
                pltpu.async_copy(table_hbm.at[idx_v.at[j0 + b]], bufs[b], gsem)
            for b in range(_NBUF):
                pltpu.make_async_copy(table_hbm.at[idx_v.at[j0 + b]],
                                      bufs[b], gsem).wait()
            for b in range(_NBUF):
                pltpu.async_copy(bufs[b], out_hbm.at[base + j0 + b], osem)
            for b in range(_NBUF):
                pltpu.make_async_copy(bufs[b], out_hbm.at[base + j0 + b],
                                      osem).wait()
            return 0

        lax.fori_loop(0, cpw // _NBUF, round_, 0)

    out = k(table, idx2d)
    return out.reshape(E, F)


def _edge_body(a_ref, g_ref, w2_ref, b2_ref, out_ref):
    A = a_ref[0]
    w2 = w2_ref[...]
    b2 = b2_ref[0]
    acc = None
    for k in range(K_):
        h = _leaky(A + g_ref[0, k])
        e = _leaky(jnp.dot(h, w2, preferred_element_type=jnp.float32) + b2)
        acc = e if acc is None else jnp.maximum(acc, e)
    out_ref[0] = acc


def _edge(a, g, w2, b2):
    B = a.shape[0]
    F = a.shape[2]
    Fo = w2.shape[1]
    return pl.pallas_call(
        _edge_body,
        grid=(B,),
        in_specs=[
            pl.BlockSpec((1, NP_, F), lambda b: (b, 0, 0)),
            pl.BlockSpec((1, K_, NP_, F), lambda b: (b, 0, 0, 0)),
            pl.BlockSpec((F, Fo), lambda b: (0, 0)),
            pl.BlockSpec((1, Fo), lambda b: (0, 0)),
        ],
        out_specs=pl.BlockSpec((1, NP_, Fo), lambda b: (b, 0, 0)),
        out_shape=jax.ShapeDtypeStruct((B, NP_, Fo), jnp.float32),
    )(a, g, w2, b2)


def _head_body(comb_ref, lw1_ref, lb1_ref, lw2_ref, lb2_ref,
               mw1_ref, mb1_ref, mw2_ref, mb2_ref, out_ref):
    comb = comb_ref[0]
    h = _leaky(jnp.dot(comb, lw1_ref[...], preferred_element_type=jnp.float32)
               + lb1_ref[0])
    h = jnp.dot(h, lw2_ref[...], preferred_element_type=jnp.float32) + lb2_ref[0]
    p = _leaky(jnp.max(h, axis=0, keepdims=True))
    o = _leaky(jnp.dot(p, mw1_ref[...], preferred_element_type=jnp.float32)
               + mb1_ref[0])
    out_ref[0] = jnp.dot(o, mw2_ref[...], preferred_element_type=jnp.float32) \
        + mb2_ref[0]


def _head(comb, lw1, lb1, lw2, lb2, mw1, mb1, mw2, mb2):
    B = comb.shape[0]
    D = comb.shape[2]
    H1 = lw1.shape[1]
    H2 = lw2.shape[1]
    H3 = mw1.shape[1]
    CLA = mw2.shape[1]
    return pl.pallas_call(
        _head_body,
        grid=(B,),
        in_specs=[
            pl.BlockSpec((1, NP_, D), lambda b: (b, 0, 0)),
            pl.BlockSpec((D, H1), lambda b: (0, 0)),
            pl.BlockSpec((1, H1), lambda b: (0, 0)),
            pl.BlockSpec((H1, H2), lambda b: (0, 0)),
            pl.BlockSpec((1, H2), lambda b: (0, 0)),
            pl.BlockSpec((H2, H3), lambda b: (0, 0)),
            pl.BlockSpec((1, H3), lambda b: (0, 0)),
            pl.BlockSpec((H3, CLA), lambda b: (0, 0)),
            pl.BlockSpec((1, CLA), lambda b: (0, 0)),
        ],
        out_specs=pl.BlockSpec((1, 1, CLA), lambda b: (b, 0, 0)),
        out_shape=jax.ShapeDtypeStruct((B, 1, CLA), jnp.float32),
    )(comb, lw1, lb1, lw2, lb2, mw1, mb1, mw2, mb2)[:, 0, :]


def _edge_conv(xin, W1, b1, W2, b2):
    B, _, d = xin.shape
    F = W1.shape[1]
    w1a = W1[:d] - W1[d:]
    w1b = W1[d:]
    idx, a, bm = _prep(xin, w1a, w1b, b1.reshape(1, F))
    g = _gather_rows(bm.reshape(B * NP_, 128), idx.reshape(B * K_ * NP_))
    g = g.reshape(B, K_, NP_, 128)
    w2p = jnp.pad(W2, ((0, 128 - F), (0, 0))) if F < 128 else W2
    return _edge(a, g, w2p, b2.reshape(1, -1))


def kernel(x, pos, batch, tq, c1_W1, c1_b1, c1_W2, c1_b2,
           c2_W1, c2_b1, c2_W2, c2_b2,
           l1_W1, l1_b1, l1_W2, l1_b2,
           m_W1, m_b1, m_W2, m_b2):
    N = x.shape[0]
    B = N // NP_
    xx = jnp.concatenate([x, pos], axis=1).reshape(B, NP_, 4)
    x1 = _edge_conv(xx, c1_W1, c1_b1, c1_W2, c1_b2)
    x2 = _edge_conv(x1, c2_W1, c2_b1, c2_W2, c2_b2)
    comb = jnp.concatenate([xx, x1, x2], axis=-1)
    return _head(comb, l1_W1, l1_b1.reshape(1, -1), l1_W2, l1_b2.reshape(1, -1),
                 m_W1, m_b1.reshape(1, -1), m_W2, m_b2.reshape(1, -1))


# speedup vs baseline: 15.0593x; 1.0087x over previous
import functools
import jax
import jax.numpy as jnp
from jax import lax
from jax.experimental import pallas as pl
from jax.experimental.pallas import tpu as pltpu
from jax.experimental.pallas import tpu_sc as plsc

NP_ = 1024
K_ = 16


def _leaky(v):
    return jnp.where(v >= 0, v, 0.01 * v)


def _prep_body(xin_ref, w1a_ref, w1b_ref, b1_ref, idx_ref, a_ref, bm_ref):
    b = pl.program_id(0)
    x = xin_ref[0]
    F = w1a_ref.shape[1]
    pad = jnp.zeros((NP_, 128 - F), jnp.float32) if F < 128 else None
    a = jnp.dot(x, w1a_ref[...], preferred_element_type=jnp.float32) + b1_ref[0]
    bm = jnp.dot(x, w1b_ref[...], preferred_element_type=jnp.float32)
    if pad is not None:
        a = jnp.concatenate([a, pad], axis=1)
        bm = jnp.concatenate([bm, pad], axis=1)
    a_ref[0] = a
    bm_ref[0] = bm
    sq = jnp.sum(x * x, axis=1)
    xxt = lax.dot_general(x, x, (((1,), (1,)), ((), ())),
                          preferred_element_type=jnp.float32)
    d2 = sq[:, None] + sq[None, :] - 2.0 * xxt
    iota = lax.broadcasted_iota(jnp.int32, (NP_, NP_), 1)
    base = b * NP_
    cols = []
    for _ in range(K_):
        m = jnp.min(d2, axis=1)
        cmp = d2 <= m[:, None]
        idxsel = jnp.min(jnp.where(cmp, iota, NP_), axis=1)
        cols.append(idxsel)
        d2 = jnp.where(cmp, jnp.float32(jnp.inf), d2)
    idx_ref[0] = jnp.stack(cols, axis=0) + base


def _prep(xin, w1a, w1b, b1):
    B = xin.shape[0]
    d = xin.shape[2]
    F = w1a.shape[1]
    return pl.pallas_call(
        _prep_body,
        grid=(B,),
        in_specs=[
            pl.BlockSpec((1, NP_, d), lambda b: (b, 0, 0)),
            pl.BlockSpec((d, F), lambda b: (0, 0)),
            pl.BlockSpec((d, F), lambda b: (0, 0)),
            pl.BlockSpec((1, F), lambda b: (0, 0)),
        ],
        out_specs=[
            pl.BlockSpec((1, K_, NP_), lambda b: (b, 0, 0)),
            pl.BlockSpec((1, NP_, 128), lambda b: (b, 0, 0)),
            pl.BlockSpec((1, NP_, 128), lambda b: (b, 0, 0)),
        ],
        out_shape=[
            jax.ShapeDtypeStruct((B, K_, NP_), jnp.int32),
            jax.ShapeDtypeStruct((B, NP_, 128), jnp.float32),
            jax.ShapeDtypeStruct((B, NP_, 128), jnp.float32),
        ],
    )(xin, w1a, w1b, b1)


_CH = 128
_NBUF = 4


def _gather_rows(table, idx):
    R, F = table.shape
    E = idx.shape[0]
    C = E // _CH
    NW = 32
    cpw = C // NW
    idx2d = idx.reshape(C, _CH)
    mesh = plsc.VectorSubcoreMesh(core_axis_name="c", subcore_axis_name="s")

    @functools.partial(
        pl.kernel,
        out_type=jax.ShapeDtypeStruct((C, _CH, F), jnp.float32),
        mesh=mesh,
        scratch_types=[
            pltpu.VMEM((cpw, _CH), jnp.int32),
            *[pltpu.VMEM((_CH, F), jnp.float32) for _ in range(_NBUF)],
            pltpu.SemaphoreType.DMA,
            pltpu.SemaphoreType.DMA,
        ],
    )
    def k(table_hbm, idx_hbm, out_hbm, idx_v, b0, b1, b2, b3, gsem, osem):
        bufs = (b0, b1, b2, b3)
        wid = lax.axis_index("s") * 2 + lax.axis_index("c")
        base = wid * cpw
        pltpu.sync_copy(idx_hbm.at[pl.ds(base, cpw)], idx_v)

        def round_(i, _):
            j0 = i * _NBUF
            for b in range(_NBUF):
                pltpu.async_copy(table_hbm.at[idx_v.at[j0 + b]], bufs[b], gsem)
            for b in range(_NBUF):
                pltpu.make_async_copy(table_hbm.at[idx_v.at[j0 + b]],
                                      bufs[b], gsem).wait()
            for b in range(_NBUF):
                pltpu.async_copy(bufs[b], out_hbm.at[base + j0 + b], osem)
            for b in range(_NBUF):
                pltpu.make_async_copy(bufs[b], out_hbm.at[base + j0 + b],
                                      osem).wait()
            return 0

        lax.fori_loop(0, cpw // _NBUF, round_, 0)

    out = k(table, idx2d)
    return out.reshape(E, F)


def _edge_body(a_ref, g_ref, w2_ref, b2_ref, out_ref):
    A = a_ref[0]
    w2 = w2_ref[...]
    b2 = b2_ref[0]
    acc = None
    for k in range(K_):
        h = _leaky(A + g_ref[0, k])
        e = _leaky(jnp.dot(h, w2, preferred_element_type=jnp.float32) + b2)
        acc = e if acc is None else jnp.maximum(acc, e)
    out_ref[0] = acc


def _edge(a, g, w2, b2):
    B = a.shape[0]
    F = a.shape[2]
    Fo = w2.shape[1]
    return pl.pallas_call(
        _edge_body,
        grid=(B,),
        in_specs=[
            pl.BlockSpec((1, NP_, F), lambda b: (b, 0, 0)),
            pl.BlockSpec((1, K_, NP_, F), lambda b: (b, 0, 0, 0)),
            pl.BlockSpec((F, Fo), lambda b: (0, 0)),
            pl.BlockSpec((1, Fo), lambda b: (0, 0)),
        ],
        out_specs=pl.BlockSpec((1, NP_, Fo), lambda b: (b, 0, 0)),
        out_shape=jax.ShapeDtypeStruct((B, NP_, Fo), jnp.float32),
    )(a, g, w2, b2)


def _head_body(comb_ref, lw1_ref, lb1_ref, lw2_ref, lb2_ref,
               mw1_ref, mb1_ref, mw2_ref, mb2_ref, out_ref):
    comb = comb_ref[0]
    h = _leaky(jnp.dot(comb, lw1_ref[...], preferred_element_type=jnp.float32)
               + lb1_ref[0])
    h = jnp.dot(h, lw2_ref[...], preferred_element_type=jnp.float32) + lb2_ref[0]
    p = _leaky(jnp.max(h, axis=0, keepdims=True))
    o = _leaky(jnp.dot(p, mw1_ref[...], preferred_element_type=jnp.float32)
               + mb1_ref[0])
    out_ref[0] = jnp.dot(o, mw2_ref[...], preferred_element_type=jnp.float32) \
        + mb2_ref[0]


def _head(comb, lw1, lb1, lw2, lb2, mw1, mb1, mw2, mb2):
    B = comb.shape[0]
    D = comb.shape[2]
    H1 = lw1.shape[1]
    H2 = lw2.shape[1]
    H3 = mw1.shape[1]
    CLA = mw2.shape[1]
    return pl.pallas_call(
        _head_body,
        grid=(B,),
        in_specs=[
            pl.BlockSpec((1, NP_, D), lambda b: (b, 0, 0)),
            pl.BlockSpec((D, H1), lambda b: (0, 0)),
            pl.BlockSpec((1, H1), lambda b: (0, 0)),
            pl.BlockSpec((H1, H2), lambda b: (0, 0)),
            pl.BlockSpec((1, H2), lambda b: (0, 0)),
            pl.BlockSpec((H2, H3), lambda b: (0, 0)),
            pl.BlockSpec((1, H3), lambda b: (0, 0)),
            pl.BlockSpec((H3, CLA), lambda b: (0, 0)),
            pl.BlockSpec((1, CLA), lambda b: (0, 0)),
        ],
        out_specs=pl.BlockSpec((1, 1, CLA), lambda b: (b, 0, 0)),
        out_shape=jax.ShapeDtypeStruct((B, 1, CLA), jnp.float32),
    )(comb, lw1, lb1, lw2, lb2, mw1, mb1, mw2, mb2)[:, 0, :]


def _edge_conv(xin, W1, b1, W2, b2):
    B, _, d = xin.shape
    F = W1.shape[1]
    w1a = W1[:d] - W1[d:]
    w1b = W1[d:]
    idx, a, bm = _prep(xin, w1a, w1b, b1.reshape(1, F))
    g = _gather_rows(bm.reshape(B * NP_, 128), idx.reshape(B * K_ * NP_))
    g = g.reshape(B, K_, NP_, 128)
    w2p = jnp.pad(W2, ((0, 128 - F), (0, 0))) if F < 128 else W2
    return _edge(a, g, w2p, b2.reshape(1, -1))


def kernel(x, pos, batch, tq, c1_W1, c1_b1, c1_W2, c1_b2,
           c2_W1, c2_b1, c2_W2, c2_b2,
           l1_W1, l1_b1, l1_W2, l1_b2,
           m_W1, m_b1, m_W2, m_b2):
    N = x.shape[0]
    B = N // NP_
    xx = jnp.concatenate([x, pos], axis=1).reshape(B, NP_, 4)
    x1 = _edge_conv(xx, c1_W1, c1_b1, c1_W2, c1_b2)
    x2 = _edge_conv(x1, c2_W1, c2_b1, c2_W2, c2_b2)
    comb = jnp.concatenate([xx, x1, x2], axis=-1)
    return _head(comb, l1_W1, l1_b1.reshape(1, -1), l1_W2, l1_b2.reshape(1, -1),
                 m_W1, m_b1.reshape(1, -1), m_W2, m_b2.reshape(1, -1))


# speedup vs baseline: 19.1071x; 1.2688x over previous
import functools
import jax
import jax.numpy as jnp
from jax import lax
from jax.experimental import pallas as pl
from jax.experimental.pallas import tpu as pltpu
from jax.experimental.pallas import tpu_sc as plsc

NP_ = 1024
K_ = 16


def _leaky(v):
    return jnp.where(v >= 0, v, 0.01 * v)


def _prep_body(xin_ref, w1a_ref, w1b_ref, b1_ref, idx_ref, a_ref, bm_ref):
    b = pl.program_id(0)
    x = xin_ref[0]
    F = w1a_ref.shape[1]
    pad = jnp.zeros((NP_, 128 - F), jnp.float32) if F < 128 else None
    a = jnp.dot(x, w1a_ref[...], preferred_element_type=jnp.float32) + b1_ref[0]
    bm = jnp.dot(x, w1b_ref[...], preferred_element_type=jnp.float32)
    if pad is not None:
        a = jnp.concatenate([a, pad], axis=1)
        bm = jnp.concatenate([bm, pad], axis=1)
    a_ref[0] = a
    bm_ref[0] = bm
    sq = jnp.sum(x * x, axis=1)
    xxt = lax.dot_general(x, x, (((1,), (1,)), ((), ())),
                          preferred_element_type=jnp.float32)
    d2 = sq[:, None] + sq[None, :] - 2.0 * xxt
    iota = lax.broadcasted_iota(jnp.int32, (1, NP_), 1).astype(jnp.float32)
    base = b * NP_
    cols = []
    for _ in range(K_):
        m = jnp.min(d2, axis=1)
        cmp = d2 <= m[:, None]
        idxsel = jnp.min(jnp.where(cmp, iota, jnp.float32(2048.0)), axis=1)
        cols.append(idxsel)
        d2 = jnp.where(cmp, jnp.float32(jnp.inf), d2)
    idx_ref[0] = jnp.stack(cols, axis=0).astype(jnp.int32) + base


def _prep(xin, w1a, w1b, b1):
    B = xin.shape[0]
    d = xin.shape[2]
    F = w1a.shape[1]
    return pl.pallas_call(
        _prep_body,
        grid=(B,),
        in_specs=[
            pl.BlockSpec((1, NP_, d), lambda b: (b, 0, 0)),
            pl.BlockSpec((d, F), lambda b: (0, 0)),
            pl.BlockSpec((d, F), lambda b: (0, 0)),
            pl.BlockSpec((1, F), lambda b: (0, 0)),
        ],
        out_specs=[
            pl.BlockSpec((1, K_, NP_), lambda b: (b, 0, 0)),
            pl.BlockSpec((1, NP_, 128), lambda b: (b, 0, 0)),
            pl.BlockSpec((1, NP_, 128), lambda b: (b, 0, 0)),
        ],
        out_shape=[
            jax.ShapeDtypeStruct((B, K_, NP_), jnp.int32),
            jax.ShapeDtypeStruct((B, NP_, 128), jnp.float32),
            jax.ShapeDtypeStruct((B, NP_, 128), jnp.float32),
        ],
    )(xin, w1a, w1b, b1)


_CH = 128
_NBUF = 4


def _gather_rows(table, idx):
    R, F = table.shape
    E = idx.shape[0]
    C = E // _CH
    NW = 32
    cpw = C // NW
    idx2d = idx.reshape(C, _CH)
    mesh = plsc.VectorSubcoreMesh(core_axis_name="c", subcore_axis_name="s")

    @functools.partial(
        pl.kernel,
        out_type=jax.ShapeDtypeStruct((C, _CH, F), jnp.float32),
        mesh=mesh,
        scratch_types=[
            pltpu.VMEM((cpw, _CH), jnp.int32),
            *[pltpu.VMEM((_CH, F), jnp.float32) for _ in range(_NBUF)],
            pltpu.SemaphoreType.DMA,
            pltpu.SemaphoreType.DMA,
        ],
    )
    def k(table_hbm, idx_hbm, out_hbm, idx_v, b0, b1, b2, b3, gsem, osem):
        bufs = (b0, b1, b2, b3)
        wid = lax.axis_index("s") * 2 + lax.axis_index("c")
        base = wid * cpw
        pltpu.sync_copy(idx_hbm.at[pl.ds(base, cpw)], idx_v)

        def round_(i, _):
            j0 = i * _NBUF
            for b in range(_NBUF):
                pltpu.async_copy(table_hbm.at[idx_v.at[j0 + b]], bufs[b], gsem)
            for b in range(_NBUF):
                pltpu.make_async_copy(table_hbm.at[idx_v.at[j0 + b]],
                                      bufs[b], gsem).wait()
            for b in range(_NBUF):
                pltpu.async_copy(bufs[b], out_hbm.at[base + j0 + b], osem)
            for b in range(_NBUF):
                pltpu.make_async_copy(bufs[b], out_hbm.at[base + j0 + b],
                                      osem).wait()
            return 0

        lax.fori_loop(0, cpw // _NBUF, round_, 0)

    out = k(table, idx2d)
    return out.reshape(E, F)


def _edge_body(a_ref, g_ref, w2_ref, b2_ref, out_ref):
    A = a_ref[0]
    w2 = w2_ref[...]
    b2 = b2_ref[0]
    acc = None
    for k in range(K_):
        h = _leaky(A + g_ref[0, k])
        e = _leaky(jnp.dot(h, w2, preferred_element_type=jnp.float32) + b2)
        acc = e if acc is None else jnp.maximum(acc, e)
    out_ref[0] = acc


def _edge(a, g, w2, b2):
    B = a.shape[0]
    F = a.shape[2]
    Fo = w2.shape[1]
    return pl.pallas_call(
        _edge_body,
        grid=(B,),
        in_specs=[
            pl.BlockSpec((1, NP_, F), lambda b: (b, 0, 0)),
            pl.BlockSpec((1, K_, NP_, F), lambda b: (b, 0, 0, 0)),
            pl.BlockSpec((F, Fo), lambda b: (0, 0)),
            pl.BlockSpec((1, Fo), lambda b: (0, 0)),
        ],
        out_specs=pl.BlockSpec((1, NP_, Fo), lambda b: (b, 0, 0)),
        out_shape=jax.ShapeDtypeStruct((B, NP_, Fo), jnp.float32),
    )(a, g, w2, b2)


def _head_body(xx_ref, x1_ref, x2_ref, lw1a_ref, lw1b_ref, lw1c_ref, lb1_ref,
               lw2_ref, lb2_ref, mw1_ref, mb1_ref, mw2_ref, mb2_ref, out_ref):
    h = (jnp.dot(xx_ref[0], lw1a_ref[...], preferred_element_type=jnp.float32)
         + jnp.dot(x1_ref[0], lw1b_ref[...], preferred_element_type=jnp.float32)
         + jnp.dot(x2_ref[0], lw1c_ref[...], preferred_element_type=jnp.float32)
         + lb1_ref[0])
    h = _leaky(h)
    h = jnp.dot(h, lw2_ref[...], preferred_element_type=jnp.float32) + lb2_ref[0]
    p = _leaky(jnp.max(h, axis=0, keepdims=True))
    o = _leaky(jnp.dot(p, mw1_ref[...], preferred_element_type=jnp.float32)
               + mb1_ref[0])
    out_ref[0] = jnp.dot(o, mw2_ref[...], preferred_element_type=jnp.float32) \
        + mb2_ref[0]


def _head(xx, x1, x2, lw1, lb1, lw2, lb2, mw1, mb1, mw2, mb2):
    B, _, d0 = xx.shape
    F1 = x1.shape[2]
    H1 = lw1.shape[1]
    H2 = lw2.shape[1]
    H3 = mw1.shape[1]
    CLA = mw2.shape[1]
    lw1a = lw1[:d0]
    lw1b = lw1[d0:d0 + F1]
    lw1c = lw1[d0 + F1:]
    F2 = lw1c.shape[0]
    return pl.pallas_call(
        _head_body,
        grid=(B,),
        in_specs=[
            pl.BlockSpec((1, NP_, d0), lambda b: (b, 0, 0)),
            pl.BlockSpec((1, NP_, F1), lambda b: (b, 0, 0)),
            pl.BlockSpec((1, NP_, F2), lambda b: (b, 0, 0)),
            pl.BlockSpec((d0, H1), lambda b: (0, 0)),
            pl.BlockSpec((F1, H1), lambda b: (0, 0)),
            pl.BlockSpec((F2, H1), lambda b: (0, 0)),
            pl.BlockSpec((1, H1), lambda b: (0, 0)),
            pl.BlockSpec((H1, H2), lambda b: (0, 0)),
            pl.BlockSpec((1, H2), lambda b: (0, 0)),
            pl.BlockSpec((H2, H3), lambda b: (0, 0)),
            pl.BlockSpec((1, H3), lambda b: (0, 0)),
            pl.BlockSpec((H3, CLA), lambda b: (0, 0)),
            pl.BlockSpec((1, CLA), lambda b: (0, 0)),
        ],
        out_specs=pl.BlockSpec((1, 1, CLA), lambda b: (b, 0, 0)),
        out_shape=jax.ShapeDtypeStruct((B, 1, CLA), jnp.float32),
    )(xx, x1, x2, lw1a, lw1b, lw1c, lb1, lw2, lb2, mw1, mb1, mw2, mb2)[:, 0, :]


def _split_w(W1, d, b1):
    return W1[:d] - W1[d:], W1[d:], b1.reshape(1, -1)


def _conv_prep(xin, w1a, w1b, b1r):
    idx, a, bm = _prep(xin, w1a, w1b, b1r)
    B = xin.shape[0]
    g = _gather_rows(bm.reshape(B * NP_, 128), idx.reshape(B * K_ * NP_))
    return a, g.reshape(B, K_, NP_, 128)


def kernel(x, pos, batch, tq, c1_W1, c1_b1, c1_W2, c1_b2,
           c2_W1, c2_b1, c2_W2, c2_b2,
           l1_W1, l1_b1, l1_W2, l1_b2,
           m_W1, m_b1, m_W2, m_b2):
    N = x.shape[0]
    B = N // NP_
    xx = jnp.concatenate([x, pos], axis=1).reshape(B, NP_, 4)
    w1a1, w1b1, b1r1 = _split_w(c1_W1, 4, c1_b1)
    w2p1 = jnp.pad(c1_W2, ((0, 64), (0, 0)))
    w1a2, w1b2, b1r2 = _split_w(c2_W1, 64, c2_b1)
    b2r1 = c1_b2.reshape(1, -1)
    b2r2 = c2_b2.reshape(1, -1)

    H = B // 2
    halves = [xx[:H], xx[H:]]
    ag1 = [None, None]
    x1 = [None, None]
    ag2 = [None, None]
    x2 = [None, None]
    for h in range(2):
        ag1[h] = _conv_prep(halves[h], w1a1, w1b1, b1r1)
    for h in range(2):
        x1[h] = _edge(ag1[h][0], ag1[h][1], w2p1, b2r1)
        ag2[h] = _conv_prep(x1[h], w1a2, w1b2, b1r2)
    for h in range(2):
        x2[h] = _edge(ag2[h][0], ag2[h][1], c2_W2, b2r2)

    x1f = jnp.concatenate(x1, axis=0)
    x2f = jnp.concatenate(x2, axis=0)
    return _head(xx, x1f, x2f, l1_W1, l1_b1.reshape(1, -1),
                 l1_W2, l1_b2.reshape(1, -1),
                 m_W1, m_b1.reshape(1, -1), m_W2, m_b2.reshape(1, -1))
